# baseline (speedup 1.0000x reference)
import functools

import jax
import jax.numpy as jnp
from jax.experimental import pallas as pl
from jax.experimental.pallas import tpu as pltpu

TILE_R = 6272


def _mm_kernel(em_ref, x_ref, o_ref):
    o_ref[...] = jax.lax.dot_general(
        em_ref[...], x_ref[...],
        dimension_numbers=(((1,), (1,)), ((), ())),
        preferred_element_type=jnp.float32,
    )


@functools.partial(jax.jit, static_argnames=())
def kernel(inputs, targets, em):
    del targets
    m, k = inputs.shape
    n = em.shape[0]
    out_t = pl.pallas_call(
        _mm_kernel,
        grid=(pl.cdiv(n, TILE_R),),
        in_specs=[
            pl.BlockSpec((TILE_R, k), lambda i: (i, 0)),
            pl.BlockSpec((m, k), lambda i: (0, 0)),
        ],
        out_specs=pl.BlockSpec((TILE_R, m), lambda i: (i, 0)),
        out_shape=jax.ShapeDtypeStruct((n, m), jnp.float32),
        compiler_params=pltpu.CompilerParams(
            dimension_semantics=("arbitrary",),
        ),
    )(em, inputs)
    return out_t.T
